# Initial kernel scaffold; baseline (speedup 1.0000x reference)
import jax
import jax.numpy as jnp
from jax.experimental import pallas as pl


def kernel(x, w_gate, W1, b1, gamma, beta, W2, b2):
    raise NotImplementedError("write your pallas kernel here")


# speedup vs baseline: 1.5295x; 1.5295x over previous
import functools

import jax
import jax.numpy as jnp
from jax import lax
from jax.experimental import pallas as pl
from jax.experimental.pallas import tpu as pltpu
from jax.experimental.pallas import tpu_sc as plsc

E = 8
K = 2
D = 2048
H = 1024
N = 2048

BLK = 128
P_PAD = N * K + E * BLK
NB = P_PAD // BLK

NC = 2
NS = 16
NW = NC * NS

_RB = 256


def _router_body(x_ref, wg_ref, logits_ref, idx_ref, wts_ref):
    logits = jnp.dot(x_ref[...], wg_ref[...], preferred_element_type=jnp.float32)
    logits_ref[...] = logits
    iota = lax.broadcasted_iota(jnp.int32, (_RB, E), 1)
    m1 = jnp.max(logits, axis=1, keepdims=True)
    i1 = jnp.min(jnp.where(logits == m1, iota, E), axis=1, keepdims=True)
    masked = jnp.where(iota == i1, -jnp.inf, logits)
    m2 = jnp.max(masked, axis=1, keepdims=True)
    i2 = jnp.min(jnp.where(masked == m2, iota, E), axis=1, keepdims=True)
    d = jnp.exp(m2 - m1)
    w1 = 1.0 / (1.0 + d)
    idx_ref[...] = jnp.concatenate([i1, i2], axis=1)
    wts_ref[...] = jnp.concatenate([w1, d * w1], axis=1)


def _router(x, w_gate):
    return pl.pallas_call(
        _router_body,
        grid=(N // _RB,),
        in_specs=[
            pl.BlockSpec((_RB, D), lambda i: (i, 0)),
            pl.BlockSpec((D, E), lambda i: (0, 0)),
        ],
        out_specs=[
            pl.BlockSpec((_RB, E), lambda i: (i, 0)),
            pl.BlockSpec((_RB, K), lambda i: (i, 0)),
            pl.BlockSpec((_RB, K), lambda i: (i, 0)),
        ],
        out_shape=[
            jax.ShapeDtypeStruct((N, E), jnp.float32),
            jax.ShapeDtypeStruct((N, K), jnp.int32),
            jax.ShapeDtypeStruct((N, K), jnp.float32),
        ],
    )(x, w_gate)


def _ffn_body(be_ref, x_ref, w1_ref, b1_ref, g_ref, bt_ref, w2_ref, b2_ref,
              wrow_ref, y_ref):
    h = jnp.dot(x_ref[...], w1_ref[0], preferred_element_type=jnp.float32)
    h = h + b1_ref[0]
    mu = jnp.mean(h, axis=1, keepdims=True)
    var = jnp.mean(jnp.square(h - mu), axis=1, keepdims=True)
    hn = (h - mu) * lax.rsqrt(var + 1e-5) * g_ref[0] + bt_ref[0]
    act = 0.5 * hn * (1.0 + lax.erf(hn * 0.7071067811865476))
    y = jnp.dot(act, w2_ref[0], preferred_element_type=jnp.float32)
    y = (y + b2_ref[0]) * wrow_ref[...]
    y_ref[...] = y


def _ffn(block_expert, x_sorted, w_sorted, W1, b1, gamma, beta, W2, b2):
    grid_spec = pltpu.PrefetchScalarGridSpec(
        num_scalar_prefetch=1,
        grid=(NB,),
        in_specs=[
            pl.BlockSpec((BLK, D), lambda i, be: (i, 0)),
            pl.BlockSpec((1, D, H), lambda i, be: (be[i], 0, 0)),
            pl.BlockSpec((1, 1, H), lambda i, be: (be[i], 0, 0)),
            pl.BlockSpec((1, 1, H), lambda i, be: (be[i], 0, 0)),
            pl.BlockSpec((1, 1, H), lambda i, be: (be[i], 0, 0)),
            pl.BlockSpec((1, H, D), lambda i, be: (be[i], 0, 0)),
            pl.BlockSpec((1, 1, D), lambda i, be: (be[i], 0, 0)),
            pl.BlockSpec((BLK, 1), lambda i, be: (i, 0)),
        ],
        out_specs=pl.BlockSpec((BLK, D), lambda i, be: (i, 0)),
    )
    return pl.pallas_call(
        _ffn_body,
        grid_spec=grid_spec,
        out_shape=jax.ShapeDtypeStruct((P_PAD, D), jnp.float32),
        compiler_params=pltpu.CompilerParams(
            dimension_semantics=("arbitrary",),
        ),
    )(block_expert, x_sorted, W1, b1.reshape(E, 1, H), gamma.reshape(E, 1, H),
      beta.reshape(E, 1, H), W2, b2.reshape(E, 1, D),
      w_sorted.reshape(P_PAD, 1))


_G_PER_W = P_PAD // NW
_G_CH = 16
_G_NCH = _G_PER_W // _G_CH


def _gather_body(x_hbm, tok_hbm, out_hbm, idx_v, bufs, sems):
    wid = lax.axis_index("s") * NC + lax.axis_index("c")
    base = wid * _G_PER_W
    pltpu.sync_copy(tok_hbm.at[pl.ds(base, _G_PER_W)], idx_v)
    pending = [None, None]
    pending[0] = pltpu.async_copy(
        x_hbm.at[idx_v.at[pl.ds(0, _G_CH)]], bufs[0], sems[0])
    for c in range(_G_NCH):
        cur = c % 2
        nxt = (c + 1) % 2
        if c + 1 < _G_NCH:
            pending[nxt] = pltpu.async_copy(
                x_hbm.at[idx_v.at[pl.ds((c + 1) * _G_CH, _G_CH)]],
                bufs[nxt], sems[nxt])
        pending[cur].wait()
        pltpu.sync_copy(bufs[cur], out_hbm.at[pl.ds(base + c * _G_CH, _G_CH)])


def _gather(x, tok_sorted):
    mesh = plsc.VectorSubcoreMesh(core_axis_name="c", subcore_axis_name="s")

    @functools.partial(
        pl.kernel,
        mesh=mesh,
        out_type=jax.ShapeDtypeStruct((P_PAD, D), jnp.float32),
        scratch_types=[
            pltpu.VMEM((_G_PER_W,), jnp.int32),
            [pltpu.VMEM((_G_CH, D), jnp.float32),
             pltpu.VMEM((_G_CH, D), jnp.float32)],
            [pltpu.SemaphoreType.DMA, pltpu.SemaphoreType.DMA],
        ],
    )
    def run(x_hbm, tok_hbm, out_hbm, idx_v, bufs, sems):
        _gather_body(x_hbm, tok_hbm, out_hbm, idx_v, bufs, sems)

    return run(x, tok_sorted)


_C_PER_W = N // NW
_C_CH = 16
_C_NCH = _C_PER_W // _C_CH


def _combine_body(y_hbm, p0_hbm, p1_hbm, out_hbm, i0_v, i1_v, a_v, b_v,
                  sem_a, sem_b):
    wid = lax.axis_index("s") * NC + lax.axis_index("c")
    base = wid * _C_PER_W
    pltpu.sync_copy(p0_hbm.at[pl.ds(base, _C_PER_W)], i0_v)
    pltpu.sync_copy(p1_hbm.at[pl.ds(base, _C_PER_W)], i1_v)
    for c in range(_C_NCH):
        cp_a = pltpu.async_copy(
            y_hbm.at[i0_v.at[pl.ds(c * _C_CH, _C_CH)]], a_v, sem_a)
        cp_b = pltpu.async_copy(
            y_hbm.at[i1_v.at[pl.ds(c * _C_CH, _C_CH)]], b_v, sem_b)
        cp_a.wait()
        cp_b.wait()

        def row_add(r, _):
            for v in range(D // 16):
                sl = pl.ds(v * 16, 16)
                a_v[r, sl] = a_v[r, sl] + b_v[r, sl]
            return 0

        lax.fori_loop(0, _C_CH, row_add, 0)
        pltpu.sync_copy(a_v, out_hbm.at[pl.ds(base + c * _C_CH, _C_CH)])


def _combine(y, pos0, pos1):
    mesh = plsc.VectorSubcoreMesh(core_axis_name="c", subcore_axis_name="s")

    @functools.partial(
        pl.kernel,
        mesh=mesh,
        out_type=jax.ShapeDtypeStruct((N, D), jnp.float32),
        scratch_types=[
            pltpu.VMEM((_C_PER_W,), jnp.int32),
            pltpu.VMEM((_C_PER_W,), jnp.int32),
            pltpu.VMEM((_C_CH, D), jnp.float32),
            pltpu.VMEM((_C_CH, D), jnp.float32),
            pltpu.SemaphoreType.DMA,
            pltpu.SemaphoreType.DMA,
        ],
    )
    def run(y_hbm, p0_hbm, p1_hbm, out_hbm, i0_v, i1_v, a_v, b_v, sem_a, sem_b):
        _combine_body(y_hbm, p0_hbm, p1_hbm, out_hbm, i0_v, i1_v, a_v, b_v,
                      sem_a, sem_b)

    return run(y, pos0, pos1)


def kernel(x, w_gate, W1, b1, gamma, beta, W2, b2):
    logits, top_idx, top_wts = _router(x, w_gate)

    e_flat = top_idx.reshape(-1)
    w_flat = top_wts.reshape(-1)
    tok_flat = jnp.repeat(jnp.arange(N, dtype=jnp.int32), K)
    onehot = (e_flat[:, None] == jnp.arange(E, dtype=jnp.int32)[None, :])
    onehot = onehot.astype(jnp.int32)
    cum = jnp.cumsum(onehot, axis=0)
    counts = cum[-1]
    rank = jnp.take_along_axis(cum - onehot, e_flat[:, None], axis=1)[:, 0]
    padded = ((counts + BLK - 1) // BLK) * BLK
    pad_start = jnp.cumsum(padded) - padded
    pos_flat = (pad_start[e_flat] + rank).astype(jnp.int32)
    tok_sorted = jnp.zeros((P_PAD,), jnp.int32).at[pos_flat].set(tok_flat)
    w_sorted = jnp.zeros((P_PAD,), jnp.float32).at[pos_flat].set(w_flat)
    pos = pos_flat.reshape(N, K)
    nb_start = pad_start // BLK
    block_expert = (jnp.searchsorted(nb_start, jnp.arange(NB), side="right")
                    - 1).astype(jnp.int32)

    x_sorted = _gather(x, tok_sorted)
    y = _ffn(block_expert, x_sorted, w_sorted, W1, b1, gamma, beta, W2, b2)
    moe_output = _combine(y, pos[:, 0], pos[:, 1])
    return (moe_output, logits, top_idx)
